# Initial kernel scaffold; baseline (speedup 1.0000x reference)
import jax
import jax.numpy as jnp
from jax.experimental import pallas as pl


def kernel(sparse_inputs, dense_inputs, seq0, seq1, T1, T2, Tseq0, Tseq1, Wd, bd, W1, b1, W2, b2, W3, b3, W4, b4, Wf, bf, Wl, bl):
    raise NotImplementedError("write your pallas kernel here")


# speedup vs baseline: 5.3756x; 5.3756x over previous
import functools

import jax
import jax.numpy as jnp
from jax import lax
from jax.experimental import pallas as pl
from jax.experimental.pallas import tpu as pltpu
from jax.experimental.pallas import tpu_sc as plsc

B = 16384
V = 100000
D = 16
NS = 26
ND = 13
L = 50
H = 200
HP = 256
NW = 32
SAMP_W = B // NW
CH1 = 64
NI1 = CH1 * NS
CH2 = 64
NI2 = CH2 * L


def _sc_body(idx_hbm, t1_hbm, t2_hbm, ts0_hbm, ts1_hbm, sq0_hbm, sq1_hbm,
             emb_out, t1_out, pool_out,
             idx1_v, rows1_v, t1_v, idx2_v, rows2_v, pool_v, sem, sem2):
    c = lax.axis_index("c")
    s = lax.axis_index("s")
    wid = s * 2 + c
    base = wid * SAMP_W

    @pl.loop(0, SAMP_W // CH1)
    def _(ci):
        o = (base + ci * CH1) * NS
        pltpu.sync_copy(idx_hbm.at[pl.ds(o, NI1)], idx1_v)
        g1 = pltpu.async_copy(t2_hbm.at[idx1_v], rows1_v, sem)
        g2 = pltpu.async_copy(t1_hbm.at[idx1_v], t1_v, sem2)
        g1.wait()
        pltpu.sync_copy(rows1_v, emb_out.at[pl.ds(o, NI1)])
        g2.wait()
        pltpu.sync_copy(t1_v, t1_out.at[pl.ds(o, NI1)])

    @pl.loop(0, SAMP_W // CH2)
    def _(ci):
        s0 = base + ci * CH2
        o = s0 * L

        def pool_table(seq_hbm, table_hbm, col0):
            pltpu.sync_copy(seq_hbm.at[pl.ds(o, NI2)], idx2_v)
            pltpu.async_copy(table_hbm.at[idx2_v], rows2_v, sem).wait()

            @pl.loop(0, CH2)
            def _(g):
                b0 = g * L
                accs = [rows2_v[b0 + r, :] for r in range(4)]
                for l in range(4, L):
                    accs[l % 4] = accs[l % 4] + rows2_v[b0 + l, :]
                acc = (accs[0] + accs[1]) + (accs[2] + accs[3])
                pool_v[g, col0:col0 + D] = acc * jnp.float32(1.0 / L)

        pool_table(sq0_hbm, ts0_hbm, 0)
        pool_table(sq1_hbm, ts1_hbm, D)
        pltpu.sync_copy(pool_v, pool_out.at[pl.ds(s0, CH2)])


def _sc_gather(flat_idx, t1f, t2f, tseq0, tseq1, s0f, s1f):
    mesh = plsc.VectorSubcoreMesh(core_axis_name="c", subcore_axis_name="s")
    f32 = jnp.float32
    k = pl.kernel(
        _sc_body,
        compiler_params=pltpu.CompilerParams(use_tc_tiling_on_sc=False),
        out_type=[
            jax.ShapeDtypeStruct((B * NS, D), f32),
            jax.ShapeDtypeStruct((B * NS,), f32),
            jax.ShapeDtypeStruct((B, 2 * D), f32),
        ],
        mesh=mesh,
        scratch_types=[
            pltpu.VMEM((NI1,), jnp.int32),
            pltpu.VMEM((NI1, D), f32),
            pltpu.VMEM((NI1,), f32),
            pltpu.VMEM((NI2,), jnp.int32),
            pltpu.VMEM((NI2, D), f32),
            pltpu.VMEM((CH2, 2 * D), f32),
            pltpu.SemaphoreType.DMA,
            pltpu.SemaphoreType.DMA,
        ],
    )
    return k(flat_idx, t1f, t2f, tseq0, tseq1, s0f, s1f)


def _tc_body(dns_ref, emb_ref, pool_ref, t1_ref, wd_ref, wemb_ref, wdns_ref,
             wpool_ref, w2_ref, w3_ref, w4_ref, b1_ref, b2_ref, b3_ref,
             s_ref, par_ref, fin_ref, like_ref):
    f32 = jnp.float32
    emb = emb_ref[...]
    dns = dns_ref[...]
    lin = (jnp.sum(t1_ref[...], axis=1, keepdims=True)
           + jnp.sum(dns * wd_ref[...], axis=1, keepdims=True)
           + par_ref[0:1, 0:1])
    summed = jnp.dot(emb, s_ref[...], preferred_element_type=f32)
    sqsum = jnp.dot(emb * emb, s_ref[...], preferred_element_type=f32)
    second = 0.5 * jnp.sum(summed * summed - sqsum, axis=1, keepdims=True)
    h = (jnp.dot(dns, wdns_ref[...], preferred_element_type=f32)
         + jnp.dot(emb, wemb_ref[...], preferred_element_type=f32)
         + jnp.dot(pool_ref[...], wpool_ref[...], preferred_element_type=f32)
         + b1_ref[...])
    h = jnp.maximum(h, 0.0)
    h = jnp.maximum(jnp.dot(h, w2_ref[...], preferred_element_type=f32)
                    + b2_ref[...], 0.0)
    h = jnp.maximum(jnp.dot(h, w3_ref[...], preferred_element_type=f32)
                    + b3_ref[...], 0.0)
    dnn = jnp.sum(h * w4_ref[...], axis=1, keepdims=True) + par_ref[0:1, 1:2]
    logits = lin + second + dnn
    fin_ref[...] = jax.nn.sigmoid(logits * par_ref[0:1, 2:3] + par_ref[0:1, 3:4])
    like_ref[...] = jax.nn.sigmoid(logits * par_ref[0:1, 4:5] + par_ref[0:1, 5:6])


def _tc_call(dns_p, emb2d, pool, t1m, wd_row, wemb, wdns, wpool, w2p, w3p,
             w4row, b1p, b2p, b3p, smat, params, bb=1024):
    f32 = jnp.float32
    nb = B // bb
    row = lambda ncol: pl.BlockSpec((bb, ncol), lambda i: (i, 0))
    full = lambda shp: pl.BlockSpec(shp, lambda i: (0, 0))
    return pl.pallas_call(
        _tc_body,
        grid=(nb,),
        in_specs=[
            row(16), row(NS * D), row(2 * D), row(NS),
            full((1, 16)), full((NS * D, HP)), full((16, HP)),
            full((2 * D, HP)), full((HP, HP)), full((HP, HP)),
            full((1, HP)), full((1, HP)), full((1, HP)), full((1, HP)),
            full((NS * D, D)), full((1, 128)),
        ],
        out_specs=[row(1), row(1)],
        out_shape=[jax.ShapeDtypeStruct((B, 1), f32)] * 2,
    )(dns_p, emb2d, pool, t1m, wd_row, wemb, wdns, wpool, w2p, w3p,
      w4row, b1p, b2p, b3p, smat, params)


def kernel(sparse_inputs, dense_inputs, seq0, seq1, T1, T2, Tseq0, Tseq1,
           Wd, bd, W1, b1, W2, b2, W3, b3, W4, b4, Wf, bf, Wl, bl):
    f32 = jnp.float32
    i32 = jnp.int32
    flat_idx = (sparse_inputs.astype(i32)
                + (jnp.arange(NS, dtype=i32) * V)[None, :]).reshape(-1)
    t2f = T2.reshape(NS * V, D)
    t1f = T1.reshape(NS * V)
    s0f = seq0.astype(i32).reshape(-1)
    s1f = seq1.astype(i32).reshape(-1)

    emb_flat, t1g, pool = _sc_gather(flat_idx, t1f, t2f, Tseq0, Tseq1, s0f, s1f)
    emb2d = emb_flat.reshape(B, NS * D)
    t1m = t1g.reshape(B, NS)

    dns_p = jnp.pad(dense_inputs, ((0, 0), (0, 16 - ND)))
    wd_row = jnp.pad(Wd[:, 0], (0, 16 - ND)).reshape(1, 16)
    pc = HP - H
    wdns = jnp.pad(W1[:ND], ((0, 16 - ND), (0, pc)))
    wemb = jnp.pad(W1[ND:ND + NS * D], ((0, 0), (0, pc)))
    wpool = jnp.pad(W1[ND + NS * D:], ((0, 0), (0, pc)))
    w2p = jnp.pad(W2, ((0, pc), (0, pc)))
    w3p = jnp.pad(W3, ((0, pc), (0, pc)))
    w4row = jnp.pad(W4[:, 0], (0, pc)).reshape(1, HP)
    b1p = jnp.pad(b1, (0, pc)).reshape(1, HP)
    b2p = jnp.pad(b2, (0, pc)).reshape(1, HP)
    b3p = jnp.pad(b3, (0, pc)).reshape(1, HP)
    smat = jnp.tile(jnp.eye(D, dtype=f32), (NS, 1))
    params = jnp.pad(
        jnp.stack([bd[0], b4[0], Wf[0, 0], bf[0], Wl[0, 0], bl[0]]),
        (0, 122)).reshape(1, 128)

    finish, like = _tc_call(dns_p, emb2d, pool, t1m, wd_row, wemb, wdns,
                            wpool, w2p, w3p, w4row, b1p, b2p, b3p, smat,
                            params)
    return (finish, like)
